# baseline (speedup 1.0000x reference)
import functools

import jax
import jax.numpy as jnp
from jax import lax
from jax.experimental import pallas as pl
from jax.experimental.pallas import tpu as pltpu
from jax.experimental.pallas import tpu_sc as plsc

N = 4194304
LEFT, RIGHT, BOTTOM, TOP = -0.5, 0.5, -0.5, 0.5
MIN_BIN_WIDTH = 1e-3
MIN_BIN_HEIGHT = 1e-3
MIN_DERIVATIVE = 1e-3


def _spline_elementwise(x, wraw, hraw, draw, log_fn):
    inside = jnp.logical_and(x > LEFT, x < RIGHT)
    xi = jnp.clip(x, LEFT + 1e-6, RIGHT - 1e-6)

    w = (1.0 / (1.0 + jnp.exp(-wraw))) * (1.0 - 2.0 * MIN_BIN_WIDTH) + MIN_BIN_WIDTH
    h = (1.0 / (1.0 + jnp.exp(-hraw))) * (1.0 - 2.0 * MIN_BIN_HEIGHT) + MIN_BIN_HEIGHT
    d = jnp.exp(draw) * (1.0 - MIN_DERIVATIVE) + MIN_DERIVATIVE

    in1 = xi >= (w - 0.5)
    icw = jnp.where(in1, w - 0.5, LEFT)
    ibw = jnp.where(in1, 1.0 - w, w)
    ich = jnp.where(in1, h - 0.5, BOTTOM)
    ih = jnp.where(in1, 1.0 - h, h)
    rib = 1.0 / ibw
    idelta = ih * rib
    id0 = jnp.where(in1, d, 1.0)
    id1 = jnp.where(in1, 1.0, d)

    theta = (xi - icw) * rib
    omt = 1.0 - theta
    tt = theta * omt
    num = ih * (idelta * theta * theta + id0 * tt)
    den = idelta + (1.0 + d - 2.0 * idelta) * tt
    rden = 1.0 / den
    out = ich + num * rden
    dnum = (idelta * idelta) * (
        id1 * theta * theta + 2.0 * idelta * tt + id0 * omt * omt
    )
    lad = log_fn(dnum * rden * rden)
    out = jnp.clip(out, BOTTOM, TOP)

    return jnp.where(inside, out, x), jnp.where(inside, lad, 0.0)


def _bit_log(x):
    bits = lax.bitcast_convert_type(x, jnp.int32)
    e = lax.shift_right_arithmetic(bits, 23) - 127
    m = lax.bitcast_convert_type((bits & 0x007FFFFF) | 0x3F800000, jnp.float32)
    t = m - 1.0
    p = t * (0.999010447 + t * (-0.489156847 + t * (0.283304325
        + t * (-0.130119415 + t * 0.030102625))))
    return e.astype(jnp.float32) * 0.6931472 + p


NW = 32
CHUNK = 8192
LANES = 16
N_SC = 1 * NW * CHUNK
PER_W = N_SC // NW
NCHUNK = PER_W // CHUNK


UNROLL = 4


def _sc_body(x_hbm, w_hbm, h_hbm, d_hbm, out_hbm, lad_hbm, *scr):
    bufs = (scr[0:6], scr[6:12])
    in_sems = scr[12:14]
    out_sems = scr[14:16]
    wid = lax.axis_index("s") * 2 + lax.axis_index("c")
    base = wid * PER_W
    hbm_in = (x_hbm, w_hbm, h_hbm, d_hbm)

    def issue_in(ci, bset, sem):
        off = base + ci * CHUNK
        return [pltpu.async_copy(h.at[pl.ds(off, CHUNK)], v, sem)
                for h, v in zip(hbm_in, bset[:4])]

    def issue_out(ci, bset, sem):
        off = base + ci * CHUNK
        return [pltpu.async_copy(bset[4], out_hbm.at[pl.ds(off, CHUNK)], sem),
                pltpu.async_copy(bset[5], lad_hbm.at[pl.ds(off, CHUNK)], sem)]

    def compute(bset):
        xv, wv, hv, dv, ov, lv = bset

        def vec_body(vi, _):
            b = vi * (LANES * UNROLL)
            for u in range(UNROLL):
                sl = pl.ds(b + u * LANES, LANES)
                o, l = _spline_elementwise(xv[sl], wv[sl], hv[sl], dv[sl],
                                           _bit_log)
                ov[sl] = o
                lv[sl] = l
            return 0

        lax.fori_loop(0, CHUNK // (LANES * UNROLL), vec_body, 0)

    in_pend = {0: issue_in(0, bufs[0], in_sems[0])}
    out_pend = {}
    for ci in range(NCHUNK):
        s = ci % 2
        if ci + 1 < NCHUNK:
            in_pend[ci + 1] = issue_in(ci + 1, bufs[(ci + 1) % 2],
                                       in_sems[(ci + 1) % 2])
        for hnd in in_pend.pop(ci):
            hnd.wait()
        if ci - 2 in out_pend:
            for hnd in out_pend.pop(ci - 2):
                hnd.wait()
        compute(bufs[s])
        out_pend[ci] = issue_out(ci, bufs[s], out_sems[s])
    for k in sorted(out_pend):
        for hnd in out_pend[k]:
            hnd.wait()


@functools.partial(
    pl.kernel,
    mesh=plsc.VectorSubcoreMesh(core_axis_name="c", subcore_axis_name="s"),
    out_type=[
        jax.ShapeDtypeStruct((N_SC,), jnp.float32),
        jax.ShapeDtypeStruct((N_SC,), jnp.float32),
    ],
    scratch_types=[pltpu.VMEM((CHUNK,), jnp.float32)] * 12
    + [pltpu.SemaphoreType.DMA] * 4,
)
def _sc_spline(*refs):
    _sc_body(*refs)


ROWS = 32768
COLS = 128
BLOCK_ROWS = 2048


def _tc_block_body(x_ref, w_ref, h_ref, d_ref, out_ref, lad_ref):
    o, l = _spline_elementwise(
        x_ref[...], w_ref[...], h_ref[...], d_ref[...], jnp.log
    )
    out_ref[...] = o
    lad_ref[...] = l


def _tc_spline(x, w, h, d, start_row=0):
    out_rows = ROWS - start_row
    bs = pl.BlockSpec((BLOCK_ROWS, COLS),
                      lambda i: (i + start_row // BLOCK_ROWS, 0))
    out, lad = pl.pallas_call(
        _tc_block_body,
        grid=(out_rows // BLOCK_ROWS,),
        in_specs=[bs] * 4,
        out_specs=[bs, bs],
        out_shape=[
            jax.ShapeDtypeStruct((ROWS, COLS), jnp.float32),
            jax.ShapeDtypeStruct((ROWS, COLS), jnp.float32),
        ],
    )(
        x.reshape(ROWS, COLS),
        w.reshape(ROWS, COLS),
        h.reshape(ROWS, COLS),
        d.reshape(ROWS, COLS),
    )
    return out, lad


def _splice_body(sc_o_ref, sc_l_ref, full_o_ref, full_l_ref,
                 out_o_ref, out_l_ref):
    out_o_ref[...] = sc_o_ref[...]
    out_l_ref[...] = sc_l_ref[...]


def _splice(sc_out, sc_lad, full_out, full_lad):
    rows_p = N_SC // COLS
    bs = pl.BlockSpec((BLOCK_ROWS, COLS), lambda i: (i, 0))
    return pl.pallas_call(
        _splice_body,
        grid=(rows_p // BLOCK_ROWS,),
        in_specs=[bs, bs, bs, bs],
        out_specs=[bs, bs],
        out_shape=[
            jax.ShapeDtypeStruct((ROWS, COLS), jnp.float32),
            jax.ShapeDtypeStruct((ROWS, COLS), jnp.float32),
        ],
        input_output_aliases={2: 0, 3: 1},
    )(sc_out.reshape(rows_p, COLS), sc_lad.reshape(rows_p, COLS),
      full_out, full_lad)


@jax.jit
def kernel(inputs_whole, width, height, derivative):
    x = inputs_whole
    w = width.reshape(N)
    h = height.reshape(N)
    d = derivative.reshape(N)
    sc_out, sc_lad = _sc_spline(x, w, h, d)
    tc_out, tc_lad = _tc_spline(x, w, h, d, start_row=N_SC // COLS)
    out, lad = _splice(sc_out, sc_lad, tc_out, tc_lad)
    return out.reshape(N), lad.reshape(N)


# speedup vs baseline: 1.2283x; 1.2283x over previous
import functools

import jax
import jax.numpy as jnp
from jax import lax
from jax.experimental import pallas as pl
from jax.experimental.pallas import tpu as pltpu
from jax.experimental.pallas import tpu_sc as plsc

N = 4194304
LEFT, RIGHT, BOTTOM, TOP = -0.5, 0.5, -0.5, 0.5
MIN_BIN_WIDTH = 1e-3
MIN_BIN_HEIGHT = 1e-3
MIN_DERIVATIVE = 1e-3


def _spline_elementwise(x, wraw, hraw, draw, log_fn):
    inside = jnp.logical_and(x > LEFT, x < RIGHT)
    xi = jnp.clip(x, LEFT + 1e-6, RIGHT - 1e-6)

    w = (1.0 / (1.0 + jnp.exp(-wraw))) * (1.0 - 2.0 * MIN_BIN_WIDTH) + MIN_BIN_WIDTH
    h = (1.0 / (1.0 + jnp.exp(-hraw))) * (1.0 - 2.0 * MIN_BIN_HEIGHT) + MIN_BIN_HEIGHT
    d = jnp.exp(draw) * (1.0 - MIN_DERIVATIVE) + MIN_DERIVATIVE

    in1 = xi >= (w - 0.5)
    icw = jnp.where(in1, w - 0.5, LEFT)
    ibw = jnp.where(in1, 1.0 - w, w)
    ich = jnp.where(in1, h - 0.5, BOTTOM)
    ih = jnp.where(in1, 1.0 - h, h)
    rib = 1.0 / ibw
    idelta = ih * rib
    id0 = jnp.where(in1, d, 1.0)
    id1 = jnp.where(in1, 1.0, d)

    theta = (xi - icw) * rib
    omt = 1.0 - theta
    tt = theta * omt
    num = ih * (idelta * theta * theta + id0 * tt)
    den = idelta + (1.0 + d - 2.0 * idelta) * tt
    rden = 1.0 / den
    out = ich + num * rden
    dnum = (idelta * idelta) * (
        id1 * theta * theta + 2.0 * idelta * tt + id0 * omt * omt
    )
    lad = log_fn(dnum * rden * rden)
    out = jnp.clip(out, BOTTOM, TOP)

    return jnp.where(inside, out, x), jnp.where(inside, lad, 0.0)


def _bit_log(x):
    bits = lax.bitcast_convert_type(x, jnp.int32)
    e = lax.shift_right_arithmetic(bits, 23) - 127
    m = lax.bitcast_convert_type((bits & 0x007FFFFF) | 0x3F800000, jnp.float32)
    t = m - 1.0
    p = t * (0.999010447 + t * (-0.489156847 + t * (0.283304325
        + t * (-0.130119415 + t * 0.030102625))))
    return e.astype(jnp.float32) * 0.6931472 + p


NW = 32
CHUNK = 8192
LANES = 16
N_SC = 1 * NW * CHUNK
PER_W = N_SC // NW
NCHUNK = PER_W // CHUNK


UNROLL = 4


def _sc_body(x_hbm, w_hbm, h_hbm, d_hbm, out_hbm, lad_hbm, *scr):
    bufs = (scr[0:6], scr[6:12])
    in_sems = scr[12:14]
    out_sems = scr[14:16]
    wid = lax.axis_index("s") * 2 + lax.axis_index("c")
    base = wid * PER_W
    hbm_in = (x_hbm, w_hbm, h_hbm, d_hbm)

    def issue_in(ci, bset, sem):
        off = base + ci * CHUNK
        return [pltpu.async_copy(h.at[pl.ds(off, CHUNK)], v, sem)
                for h, v in zip(hbm_in, bset[:4])]

    def issue_out(ci, bset, sem):
        off = base + ci * CHUNK
        return [pltpu.async_copy(bset[4], out_hbm.at[pl.ds(off, CHUNK)], sem),
                pltpu.async_copy(bset[5], lad_hbm.at[pl.ds(off, CHUNK)], sem)]

    def compute(bset):
        xv, wv, hv, dv, ov, lv = bset

        def vec_body(vi, _):
            b = vi * (LANES * UNROLL)
            for u in range(UNROLL):
                sl = pl.ds(b + u * LANES, LANES)
                o, l = _spline_elementwise(xv[sl], wv[sl], hv[sl], dv[sl],
                                           _bit_log)
                ov[sl] = o
                lv[sl] = l
            return 0

        lax.fori_loop(0, CHUNK // (LANES * UNROLL), vec_body, 0)

    in_pend = {0: issue_in(0, bufs[0], in_sems[0])}
    out_pend = {}
    for ci in range(NCHUNK):
        s = ci % 2
        if ci + 1 < NCHUNK:
            in_pend[ci + 1] = issue_in(ci + 1, bufs[(ci + 1) % 2],
                                       in_sems[(ci + 1) % 2])
        for hnd in in_pend.pop(ci):
            hnd.wait()
        if ci - 2 in out_pend:
            for hnd in out_pend.pop(ci - 2):
                hnd.wait()
        compute(bufs[s])
        out_pend[ci] = issue_out(ci, bufs[s], out_sems[s])
    for k in sorted(out_pend):
        for hnd in out_pend[k]:
            hnd.wait()


@functools.partial(
    pl.kernel,
    mesh=plsc.VectorSubcoreMesh(core_axis_name="c", subcore_axis_name="s"),
    out_type=[
        jax.ShapeDtypeStruct((N_SC,), jnp.float32),
        jax.ShapeDtypeStruct((N_SC,), jnp.float32),
    ],
    scratch_types=[pltpu.VMEM((CHUNK,), jnp.float32)] * 12
    + [pltpu.SemaphoreType.DMA] * 4,
)
def _sc_spline(*refs):
    _sc_body(*refs)


ROWS = 32768
COLS = 128
BLOCK_ROWS = 2048


def _tc_block_body(x_ref, w_ref, h_ref, d_ref, out_ref, lad_ref):
    o, l = _spline_elementwise(
        x_ref[...], w_ref[...], h_ref[...], d_ref[...], _bit_log
    )
    out_ref[...] = o
    lad_ref[...] = l


def _tc_spline(x, w, h, d, start_row=0):
    out_rows = ROWS - start_row
    bs = pl.BlockSpec((BLOCK_ROWS, COLS),
                      lambda i: (i + start_row // BLOCK_ROWS, 0))
    out, lad = pl.pallas_call(
        _tc_block_body,
        grid=(out_rows // BLOCK_ROWS,),
        in_specs=[bs] * 4,
        out_specs=[bs, bs],
        out_shape=[
            jax.ShapeDtypeStruct((ROWS, COLS), jnp.float32),
            jax.ShapeDtypeStruct((ROWS, COLS), jnp.float32),
        ],
    )(
        x.reshape(ROWS, COLS),
        w.reshape(ROWS, COLS),
        h.reshape(ROWS, COLS),
        d.reshape(ROWS, COLS),
    )
    return out, lad


def _splice_body(sc_o_ref, sc_l_ref, full_o_ref, full_l_ref,
                 out_o_ref, out_l_ref):
    out_o_ref[...] = sc_o_ref[...]
    out_l_ref[...] = sc_l_ref[...]


def _splice(sc_out, sc_lad, full_out, full_lad):
    rows_p = N_SC // COLS
    bs = pl.BlockSpec((BLOCK_ROWS, COLS), lambda i: (i, 0))
    return pl.pallas_call(
        _splice_body,
        grid=(rows_p // BLOCK_ROWS,),
        in_specs=[bs, bs, bs, bs],
        out_specs=[bs, bs],
        out_shape=[
            jax.ShapeDtypeStruct((ROWS, COLS), jnp.float32),
            jax.ShapeDtypeStruct((ROWS, COLS), jnp.float32),
        ],
        input_output_aliases={2: 0, 3: 1},
    )(sc_out.reshape(rows_p, COLS), sc_lad.reshape(rows_p, COLS),
      full_out, full_lad)


@jax.jit
def kernel(inputs_whole, width, height, derivative):
    x = inputs_whole
    w = width.reshape(N)
    h = height.reshape(N)
    d = derivative.reshape(N)
    out, lad = _tc_spline(x, w, h, d, start_row=0)
    return out.reshape(N), lad.reshape(N)
